# baseline (speedup 1.0000x reference)
import functools

import jax
import jax.numpy as jnp
from jax import lax
from jax.experimental import pallas as pl
from jax.experimental.pallas import tpu as pltpu
from jax.experimental.pallas import tpu_sc as plsc

_N = 9984
_E = 319488
_B = 78
_F = 128
_H = 256
_CH = 128
_NCH = _E // _CH
_NSC = 2
_NT = 16
_DCH = _NCH // (_NSC * _NT)
_SL = _N // _NT

_CC = 104
_ET = _E // _NT
_CT = _ET // _CC
_CG = 6
_CNG = _CT // _CG
_DW = 16

_mesh = plsc.VectorSubcoreMesh(core_axis_name="c", subcore_axis_name="s")


_DDRAIN = 8


@functools.partial(
    pl.kernel,
    out_type=jax.ShapeDtypeStruct((_NSC * _N, _F), jnp.float32),
    mesh=_mesh,
    scratch_types=[
        pltpu.VMEM((_DCH, _CH), jnp.int32),
        pltpu.VMEM((_CH, _F), jnp.float32),
        pltpu.VMEM_SHARED((_N, _F), jnp.float32),
        pltpu.SemaphoreType.DMA,
    ],
)
def _deg_sc(dstr_hbm, ones_hbm, zeros_hbm, out_hbm, idx_v, ones_v, acc, ssem):
    c = lax.axis_index("c")
    s = lax.axis_index("s")
    w = c * _NT + s
    pltpu.sync_copy(dstr_hbm.at[w], idx_v)
    pltpu.sync_copy(ones_hbm, ones_v)
    pltpu.sync_copy(zeros_hbm.at[pl.ds(s * _SL, _SL)], acc.at[pl.ds(s * _SL, _SL)])
    plsc.subcore_barrier()

    def _drain():
        pltpu.make_async_copy(ones_hbm, ones_v, ssem).wait()

    def body(j, carry):
        pltpu.async_copy(ones_v, acc.at[idx_v.at[j]], ssem, add=True)

        @pl.when(j >= _DDRAIN)
        def _():
            _drain()

        return carry

    lax.fori_loop(0, _DCH, body, 0)
    for _ in range(_DDRAIN):
        _drain()
    plsc.subcore_barrier()
    pltpu.sync_copy(acc.at[pl.ds(s * _SL, _SL)],
                    out_hbm.at[pl.ds(c * _N + s * _SL, _SL)])


@functools.partial(
    pl.kernel,
    out_type=jax.ShapeDtypeStruct((_NSC * _N, _F), jnp.float32),
    mesh=_mesh,
    scratch_types=[
        pltpu.VMEM((3, 2, _CG, _CC), jnp.int32),
        pltpu.VMEM((3, _CC, _F), jnp.float32),
        pltpu.VMEM_SHARED((_N, _F), jnp.float32),
        [pltpu.SemaphoreType.DMA] * 3,
        [pltpu.SemaphoreType.DMA] * 3,
        pltpu.SemaphoreType.DMA,
    ],
)
def _conv_sc(g_hbm, idx_hbm, out_hbm, idx_v, buf, acc, gsems, ssems, isem):
    c = lax.axis_index("c")
    s = lax.axis_index("s")
    pltpu.sync_copy(g_hbm.at[pl.ds(c * _N + s * _SL, _SL)],
                    acc.at[pl.ds(s * _SL, _SL)])
    plsc.subcore_barrier()

    def _gdrain(k):
        pltpu.make_async_copy(g_hbm.at[pl.ds(0, _CC)], buf.at[k % 3],
                              gsems[k % 3]).wait()

    def _sdrain(k):
        pltpu.make_async_copy(g_hbm.at[pl.ds(0, _CC)], buf.at[k % 3],
                              ssems[k % 3]).wait()

    def _idrain():
        pltpu.make_async_copy(idx_hbm.at[c, s, 0], idx_v.at[0], isem).wait()

    def _islot(gi):
        return lax.rem(gi, 3)

    pltpu.sync_copy(idx_hbm.at[c, s, 0], idx_v.at[0])
    pltpu.async_copy(idx_hbm.at[c, s, 1], idx_v.at[1], isem)
    pltpu.async_copy(g_hbm.at[idx_v.at[0, 0, 0]], buf.at[0], gsems[0])
    pltpu.async_copy(g_hbm.at[idx_v.at[0, 0, 1]], buf.at[1], gsems[1])

    def grp(gp, carry):
        @pl.when(gp < _CNG - 1)
        def _():
            _idrain()

        @pl.when(gp < _CNG - 2)
        def _():
            sl2 = _islot(gp + 2)
            pltpu.async_copy(idx_hbm.at[c, s, gp + 2], idx_v.at[sl2], isem)

        sl0 = _islot(gp)
        sl1 = _islot(gp + 1)
        for k in range(_CG):
            _gdrain(k)
            if k == 0:
                @pl.when(gp > 0)
                def _():
                    _sdrain(k - 1)
            else:
                _sdrain(k - 1)
            qrow = (k + 2) % _CG
            qsl = sl1 if k >= _CG - 2 else sl0
            if k >= _CG - 2:
                @pl.when(gp < _CNG - 1)
                def _():
                    pltpu.async_copy(g_hbm.at[idx_v.at[qsl, 0, qrow]],
                                     buf.at[(k + 2) % 3], gsems[(k + 2) % 3])
            else:
                pltpu.async_copy(g_hbm.at[idx_v.at[qsl, 0, qrow]],
                                 buf.at[(k + 2) % 3], gsems[(k + 2) % 3])
            pltpu.async_copy(buf.at[k % 3], acc.at[idx_v.at[sl0, 1, k]],
                             ssems[k % 3], add=True)
        return carry

    lax.fori_loop(0, _CNG, grp, 0)
    _sdrain(_CG - 1)
    plsc.subcore_barrier()
    pltpu.sync_copy(acc.at[pl.ds(s * _SL, _SL)],
                    out_hbm.at[pl.ds(c * _N + s * _SL, _SL)])


def _bn_relu(u):
    m = jnp.mean(u, axis=0)
    v = jnp.mean((u - m) ** 2, axis=0)
    return jnp.maximum((u - m) * lax.rsqrt(v + 1e-5), 0.0)


def _tc0_body(x_ref, t_ref, wt_ref, bt_ref, w1_ref, deg_ref, g_ref, dinv_ref):
    deg = (deg_ref[pl.ds(0, _N), pl.ds(0, 1)]
           + deg_ref[pl.ds(_N, _N), pl.ds(0, 1)] + 1.0)
    dinv = lax.rsqrt(deg)
    dinv_ref[...] = dinv
    z = jnp.dot(t_ref[...], wt_ref[...], preferred_element_type=jnp.float32)
    z = z + bt_ref[...][None, :]
    te = _bn_relu(z)
    ts = jnp.dot(te, w1_ref[_F:, :], preferred_element_type=jnp.float32)
    big = jnp.dot(x_ref[...], w1_ref[:_F, :], preferred_element_type=jnp.float32)
    rep = jnp.broadcast_to(ts[:, None, :], (_B, _F, _H)).reshape(_N, _H)
    g = (big + rep) * dinv
    g_ref[pl.ds(0, _N), :] = g[:, :_F]
    g_ref[pl.ds(_N, _N), :] = g[:, _F:]


def _tc_mid_body(a_ref, dinv_ref, b_ref, w_ref, g_ref):
    dinv = dinv_ref[...]
    u0 = a_ref[pl.ds(0, _N), :] * dinv + b_ref[...][None, :_F]
    u1 = a_ref[pl.ds(_N, _N), :] * dinv + b_ref[...][None, _F:]
    x0 = _bn_relu(u0)
    x1 = _bn_relu(u1)
    g = jnp.dot(x0, w_ref[:_F, :], preferred_element_type=jnp.float32)
    g = g + jnp.dot(x1, w_ref[_F:, :], preferred_element_type=jnp.float32)
    g = g * dinv
    g_ref[pl.ds(0, _N), :] = g[:, :_F]
    g_ref[pl.ds(_N, _N), :] = g[:, _F:]


def _tc_fin_body(a_ref, dinv_ref, b_ref, out_ref):
    dinv = dinv_ref[...]
    u0 = a_ref[pl.ds(0, _N), :] * dinv + b_ref[...][None, :_F]
    u1 = a_ref[pl.ds(_N, _N), :] * dinv + b_ref[...][None, _F:]
    out_ref[:, :_F] = _bn_relu(u0)
    out_ref[:, _F:] = _bn_relu(u1)


_tc0 = pl.pallas_call(
    _tc0_body,
    out_shape=[jax.ShapeDtypeStruct((_NSC * _N, _F), jnp.float32),
               jax.ShapeDtypeStruct((_N, 1), jnp.float32)],
)

_tc_mid = pl.pallas_call(
    _tc_mid_body,
    out_shape=jax.ShapeDtypeStruct((_NSC * _N, _F), jnp.float32),
)

_tc_fin = pl.pallas_call(
    _tc_fin_body,
    out_shape=jax.ShapeDtypeStruct((_N, _H), jnp.float32),
)


def kernel(x, t, edge_index, Wt, bt, W1, b1, W2, b2, W3, b3):
    src = edge_index[0]
    dst = edge_index[1]
    srcp = jnp.concatenate([src, src + _N]).reshape(_NSC, _NT, _CNG, 1, _CG, _CC)
    dstp = jnp.broadcast_to(dst.reshape(1, _NT, _CNG, 1, _CG, _CC),
                            (_NSC, _NT, _CNG, 1, _CG, _CC))
    idxc = jnp.concatenate([srcp, dstp], axis=3)

    dstr_deg = dst.reshape(_NSC * _NT, _DCH, _CH)
    ones128 = jnp.ones((_CH, _F), jnp.float32)
    zeros_tab = jnp.zeros((_N, _F), jnp.float32)
    degf = _deg_sc(dstr_deg, ones128, zeros_tab)
    g, dinv = _tc0(x, t, Wt, bt, W1, degf)
    a = _conv_sc(g, idxc)
    g = _tc_mid(a, dinv, b1, W2)
    a = _conv_sc(g, idxc)
    g = _tc_mid(a, dinv, b2, W3)
    a = _conv_sc(g, idxc)
    return _tc_fin(a, dinv, b3)


# speedup vs baseline: 1.0019x; 1.0019x over previous
import functools

import jax
import jax.numpy as jnp
from jax import lax
from jax.experimental import pallas as pl
from jax.experimental.pallas import tpu as pltpu
from jax.experimental.pallas import tpu_sc as plsc

_N = 9984
_E = 319488
_B = 78
_F = 128
_H = 256
_CH = 128
_NCH = _E // _CH
_NSC = 2
_NT = 16
_DCH = _NCH // (_NSC * _NT)
_SL = _N // _NT

_CC = 104
_ET = _E // _NT
_CT = _ET // _CC
_CG = 6
_CNG = _CT // _CG

_mesh = plsc.VectorSubcoreMesh(core_axis_name="c", subcore_axis_name="s")


_DDRAIN = 8


@functools.partial(
    pl.kernel,
    out_type=jax.ShapeDtypeStruct((_NSC * _N, _F), jnp.float32),
    mesh=_mesh,
    scratch_types=[
        pltpu.VMEM((_DCH, _CH), jnp.int32),
        pltpu.VMEM((_CH, _F), jnp.float32),
        pltpu.VMEM_SHARED((_N, _F), jnp.float32),
        pltpu.SemaphoreType.DMA,
    ],
)
def _deg_sc(dstr_hbm, ones_hbm, zeros_hbm, out_hbm, idx_v, ones_v, acc, ssem):
    c = lax.axis_index("c")
    s = lax.axis_index("s")
    w = c * _NT + s
    pltpu.sync_copy(dstr_hbm.at[w], idx_v)
    pltpu.sync_copy(ones_hbm, ones_v)
    pltpu.sync_copy(zeros_hbm.at[pl.ds(s * _SL, _SL)], acc.at[pl.ds(s * _SL, _SL)])
    plsc.subcore_barrier()

    def _drain():
        pltpu.make_async_copy(ones_hbm, ones_v, ssem).wait()

    def body(j, carry):
        pltpu.async_copy(ones_v, acc.at[idx_v.at[j]], ssem, add=True)

        @pl.when(j >= _DDRAIN)
        def _():
            _drain()

        return carry

    lax.fori_loop(0, _DCH, body, 0)
    for _ in range(_DDRAIN):
        _drain()
    plsc.subcore_barrier()
    pltpu.sync_copy(acc.at[pl.ds(s * _SL, _SL)],
                    out_hbm.at[pl.ds(c * _N + s * _SL, _SL)])


@functools.partial(
    pl.kernel,
    out_type=jax.ShapeDtypeStruct((_NSC * _N, _F), jnp.float32),
    mesh=_mesh,
    scratch_types=[
        pltpu.VMEM((3, 2, _CG, _CC), jnp.int32),
        pltpu.VMEM((3, _CC, _F), jnp.float32),
        pltpu.VMEM_SHARED((_N, _F), jnp.float32),
        [pltpu.SemaphoreType.DMA] * 3,
        [pltpu.SemaphoreType.DMA] * 3,
        pltpu.SemaphoreType.DMA,
    ],
)
def _conv_sc(g_hbm, idx_hbm, out_hbm, idx_v, buf, acc, gsems, ssems, isem):
    c = lax.axis_index("c")
    s = lax.axis_index("s")
    pltpu.sync_copy(g_hbm.at[pl.ds(c * _N + s * _SL, _SL)],
                    acc.at[pl.ds(s * _SL, _SL)])
    plsc.subcore_barrier()

    def _gdrain(k):
        pltpu.make_async_copy(g_hbm.at[pl.ds(0, _CC)], buf.at[k % 3],
                              gsems[k % 3]).wait()

    def _sdrain(k):
        pltpu.make_async_copy(g_hbm.at[pl.ds(0, _CC)], buf.at[k % 3],
                              ssems[k % 3]).wait()

    def _idrain():
        pltpu.make_async_copy(idx_hbm.at[c, s, 0], idx_v.at[0], isem).wait()

    def _islot(gi):
        return lax.rem(gi, 3)

    pltpu.sync_copy(idx_hbm.at[c, s, 0], idx_v.at[0])
    pltpu.async_copy(idx_hbm.at[c, s, 1], idx_v.at[1], isem)
    pltpu.async_copy(g_hbm.at[idx_v.at[0, 0, 0]], buf.at[0], gsems[0])
    pltpu.async_copy(g_hbm.at[idx_v.at[0, 0, 1]], buf.at[1], gsems[1])

    def grp(gp, carry):
        @pl.when(gp < _CNG - 1)
        def _():
            _idrain()

        @pl.when(gp < _CNG - 2)
        def _():
            sl2 = _islot(gp + 2)
            pltpu.async_copy(idx_hbm.at[c, s, gp + 2], idx_v.at[sl2], isem)

        sl0 = _islot(gp)
        sl1 = _islot(gp + 1)
        for k in range(_CG):
            _gdrain(k)
            if k == 0:
                @pl.when(gp > 0)
                def _():
                    _sdrain(k - 1)
            else:
                _sdrain(k - 1)
            qrow = (k + 2) % _CG
            qsl = sl1 if k >= _CG - 2 else sl0
            if k >= _CG - 2:
                @pl.when(gp < _CNG - 1)
                def _():
                    pltpu.async_copy(g_hbm.at[idx_v.at[qsl, 0, qrow]],
                                     buf.at[(k + 2) % 3], gsems[(k + 2) % 3])
            else:
                pltpu.async_copy(g_hbm.at[idx_v.at[qsl, 0, qrow]],
                                 buf.at[(k + 2) % 3], gsems[(k + 2) % 3])
            pltpu.async_copy(buf.at[k % 3], acc.at[idx_v.at[sl0, 1, k]],
                             ssems[k % 3], add=True)
        return carry

    lax.fori_loop(0, _CNG, grp, 0)
    _sdrain(_CG - 1)
    plsc.subcore_barrier()
    pltpu.sync_copy(acc.at[pl.ds(s * _SL, _SL)],
                    out_hbm.at[pl.ds(c * _N + s * _SL, _SL)])


def _bn_relu(u):
    m = jnp.mean(u, axis=0)
    v = jnp.mean((u - m) ** 2, axis=0)
    return jnp.maximum((u - m) * lax.rsqrt(v + 1e-5), 0.0)


def _tc0_body(x_ref, t_ref, wt_ref, bt_ref, w1_ref, deg_ref, g_ref, dinv_ref):
    deg = (deg_ref[pl.ds(0, _N), pl.ds(0, 1)]
           + deg_ref[pl.ds(_N, _N), pl.ds(0, 1)] + 1.0)
    dinv = lax.rsqrt(deg)
    dinv_ref[...] = dinv
    z = jnp.dot(t_ref[...], wt_ref[...], preferred_element_type=jnp.float32)
    z = z + bt_ref[...][None, :]
    te = _bn_relu(z)
    ts = jnp.dot(te, w1_ref[_F:, :], preferred_element_type=jnp.float32)
    big = jnp.dot(x_ref[...], w1_ref[:_F, :], preferred_element_type=jnp.float32)
    rep = jnp.broadcast_to(ts[:, None, :], (_B, _F, _H)).reshape(_N, _H)
    g = (big + rep) * dinv
    g_ref[pl.ds(0, _N), :] = g[:, :_F]
    g_ref[pl.ds(_N, _N), :] = g[:, _F:]


def _tc_mid_body(a_ref, dinv_ref, b_ref, w_ref, g_ref):
    dinv = dinv_ref[...]
    u0 = a_ref[pl.ds(0, _N), :] * dinv + b_ref[...][None, :_F]
    u1 = a_ref[pl.ds(_N, _N), :] * dinv + b_ref[...][None, _F:]
    x0 = _bn_relu(u0)
    x1 = _bn_relu(u1)
    g = jnp.dot(x0, w_ref[:_F, :], preferred_element_type=jnp.float32)
    g = g + jnp.dot(x1, w_ref[_F:, :], preferred_element_type=jnp.float32)
    g = g * dinv
    g_ref[pl.ds(0, _N), :] = g[:, :_F]
    g_ref[pl.ds(_N, _N), :] = g[:, _F:]


def _tc_fin_body(a_ref, dinv_ref, b_ref, out_ref):
    dinv = dinv_ref[...]
    u0 = a_ref[pl.ds(0, _N), :] * dinv + b_ref[...][None, :_F]
    u1 = a_ref[pl.ds(_N, _N), :] * dinv + b_ref[...][None, _F:]
    out_ref[:, :_F] = _bn_relu(u0)
    out_ref[:, _F:] = _bn_relu(u1)


_tc0 = pl.pallas_call(
    _tc0_body,
    out_shape=[jax.ShapeDtypeStruct((_NSC * _N, _F), jnp.float32),
               jax.ShapeDtypeStruct((_N, 1), jnp.float32)],
)

_tc_mid = pl.pallas_call(
    _tc_mid_body,
    out_shape=jax.ShapeDtypeStruct((_NSC * _N, _F), jnp.float32),
)

_tc_fin = pl.pallas_call(
    _tc_fin_body,
    out_shape=jax.ShapeDtypeStruct((_N, _H), jnp.float32),
)


def kernel(x, t, edge_index, Wt, bt, W1, b1, W2, b2, W3, b3):
    src = edge_index[0]
    dst = edge_index[1]
    srcp = jnp.concatenate([src, src + _N]).reshape(_NSC, _NT, _CNG, 1, _CG, _CC)
    dstp = jnp.broadcast_to(dst.reshape(1, _NT, _CNG, 1, _CG, _CC),
                            (_NSC, _NT, _CNG, 1, _CG, _CC))
    idxc = jnp.concatenate([srcp, dstp], axis=3)

    dstr_deg = dst.reshape(_NSC * _NT, _DCH, _CH)
    ones128 = jnp.ones((_CH, _F), jnp.float32)
    zeros_tab = jnp.zeros((_N, _F), jnp.float32)
    degf = _deg_sc(dstr_deg, ones128, zeros_tab)
    g, dinv = _tc0(x, t, Wt, bt, W1, degf)
    a = _conv_sc(g, idxc)
    g = _tc_mid(a, dinv, b1, W2)
    a = _conv_sc(g, idxc)
    g = _tc_mid(a, dinv, b2, W3)
    a = _conv_sc(g, idxc)
    return _tc_fin(a, dinv, b3)
